# baseline (speedup 1.0000x reference)
import functools

import jax
import jax.numpy as jnp
from jax import lax
from jax.experimental import pallas as pl
from jax.experimental.pallas import tpu as pltpu
from jax.experimental.pallas import tpu_sc as plsc

D_MODEL = 768
MAXLEN = 30
N_SEGMENTS = 2
VOCAB = 4
N_COMBO = VOCAB * N_SEGMENTS * MAXLEN
_EPS = 1e-5

_NC = 2
_NS = 16
_NW = _NC * _NS
_LANES = 16
_SL_PAD = 32


def _combo_body(tok_ref, pos_ref, seg_ref, gamma_ref, beta_ref, out_ref):
    row = lax.broadcasted_iota(jnp.int32, (N_COMBO, 1), 0)

    def onehot(ids, n):
        cols = lax.broadcasted_iota(jnp.int32, (N_COMBO, n), 1)
        return (ids == cols).astype(jnp.float32)

    emb = jnp.dot(onehot(row // (N_SEGMENTS * MAXLEN), VOCAB), tok_ref[...],
                  preferred_element_type=jnp.float32)
    emb = emb + jnp.dot(onehot((row // MAXLEN) % N_SEGMENTS, N_SEGMENTS),
                        seg_ref[...], preferred_element_type=jnp.float32)
    emb = emb + jnp.dot(onehot(row % MAXLEN, MAXLEN), pos_ref[...],
                        preferred_element_type=jnp.float32)
    mean = jnp.mean(emb, axis=-1, keepdims=True)
    cent = emb - mean
    var = jnp.mean(cent * cent, axis=-1, keepdims=True)
    out_ref[...] = (cent * lax.rsqrt(var + _EPS)) * gamma_ref[...] + beta_ref[...]


@functools.lru_cache(maxsize=None)
def _sc_lookup(batch: int, seq_len: int):
    assert batch % _NW == 0
    nb = batch // _NW
    nw_words = nb * _SL_PAD
    assert nb % 2 == 0
    mesh = plsc.VectorSubcoreMesh(core_axis_name="c", subcore_axis_name="s")

    @functools.partial(
        pl.kernel,
        mesh=mesh,
        out_type=jax.ShapeDtypeStruct((batch, _SL_PAD, D_MODEL), jnp.float32),
        scratch_types=[
            pltpu.VMEM((nw_words,), jnp.int32),
            pltpu.VMEM((nw_words,), jnp.int32),
            pltpu.VMEM((nw_words,), jnp.int32),
            pltpu.VMEM((2, _SL_PAD, D_MODEL), jnp.float32),
            pltpu.SemaphoreType.DMA,
            pltpu.SemaphoreType.DMA,
        ],
    )
    def body(x_hbm, s_hbm, combo_hbm, out_hbm, x_v, s_v, idx_v, buf_v, sem0, sem1):
        sems = (sem0, sem1)
        wid = lax.axis_index("s") * _NC + lax.axis_index("c")
        base_b = wid * nb
        base_w = pl.multiple_of(wid * nw_words, nw_words)
        pltpu.sync_copy(x_hbm.at[pl.ds(base_w, nw_words)], x_v)
        pltpu.sync_copy(s_hbm.at[pl.ds(base_w, nw_words)], s_v)

        lanes = lax.iota(jnp.int32, _LANES)

        def cid_body(j, carry):
            off = pl.multiple_of(j * _LANES, _LANES)
            xv = x_v[pl.ds(off, _LANES)]
            sv = s_v[pl.ds(off, _LANES)]
            t = jnp.minimum((off + lanes) % _SL_PAD, seq_len - 1)
            idx_v[pl.ds(off, _LANES)] = (xv * N_SEGMENTS + sv) * seq_len + t
            return carry

        lax.fori_loop(0, nw_words // _LANES, cid_body, 0)

        def gather_row(i, b):
            off = pl.multiple_of(i * _SL_PAD, _SL_PAD)
            return pltpu.async_copy(
                combo_hbm.at[idx_v.at[pl.ds(off, _SL_PAD)]], buf_v.at[b], sems[b])

        gather_row(0, 0)
        gather_row(1, 1)

        def outer(ii, carry):
            for b in range(2):
                i = ii * 2 + b
                off = pl.multiple_of(i * _SL_PAD, _SL_PAD)
                pltpu.make_async_copy(
                    combo_hbm.at[idx_v.at[pl.ds(off, _SL_PAD)]], buf_v.at[b], sems[b]
                ).wait()
                pltpu.sync_copy(buf_v.at[b], out_hbm.at[base_b + i])

                @pl.when(i + 2 < nb)
                def _start_next():
                    gather_row(i + 2, b)

            return carry

        lax.fori_loop(0, nb // 2, outer, 0)

    return body


def kernel(x, seg, tok_table, pos_table, seg_table, gamma, beta):
    combo = pl.pallas_call(
        _combo_body,
        out_shape=jax.ShapeDtypeStruct((N_COMBO, D_MODEL), jnp.float32),
    )(tok_table, pos_table, seg_table,
      gamma.reshape(1, D_MODEL).astype(jnp.float32),
      beta.reshape(1, D_MODEL).astype(jnp.float32))

    batch, seq_len = x.shape
    pad = _SL_PAD - seq_len
    xp = jnp.pad(x.astype(jnp.int32), ((0, 0), (0, pad))).reshape(-1)
    sp = jnp.pad(seg.astype(jnp.int32), ((0, 0), (0, pad))).reshape(-1)
    out = _sc_lookup(batch, seq_len)(xp, sp, combo)
    return out[:, :seq_len, :]
